# Initial kernel scaffold; baseline (speedup 1.0000x reference)
import jax
import jax.numpy as jnp
from jax.experimental import pallas as pl


def kernel(features, coords, search_area):
    raise NotImplementedError("write your pallas kernel here")


# speedup vs baseline: 2.7011x; 2.7011x over previous
import functools

import jax
import jax.numpy as jnp
from jax import lax
from jax.experimental import pallas as pl
from jax.experimental.pallas import tpu as pltpu
from jax.experimental.pallas import tpu_sc as plsc

XD, YD, ZD = 20, 20, 20
R = XD * YD * ZD
R_PAD = 8192
NC, NS = 2, 16
CNT_W = 16


def _prepass_body(vs_ref, coords_ref, feat_ref, idx_ref, featT_ref):
    c = coords_ref[0]
    vs = vs_ref[0]
    d = jnp.floor(c / vs)
    vi = jnp.clip(d + 10.0, 0.0, 19.0)
    flat = vi[0:1] * float(YD * ZD) + vi[1:2] * float(ZD) + vi[2:3]
    idx_ref[0] = flat.astype(jnp.int32)
    featT_ref[0] = feat_ref[0].T


def _prepass(vs, coordsT, features):
    B, C, N = features.shape
    NB = 2048
    return pl.pallas_call(
        _prepass_body,
        grid=(B, N // NB),
        in_specs=[
            pl.BlockSpec((1, 3, 1), lambda b, i: (b, 0, 0)),
            pl.BlockSpec((1, 3, NB), lambda b, i: (b, 0, i)),
            pl.BlockSpec((1, C, NB), lambda b, i: (b, 0, i)),
        ],
        out_specs=[
            pl.BlockSpec((1, 1, NB), lambda b, i: (b, 0, i)),
            pl.BlockSpec((1, NB, C), lambda b, i: (b, i, 0)),
        ],
        out_shape=[
            jax.ShapeDtypeStruct((B, 1, N), jnp.int32),
            jax.ShapeDtypeStruct((B, N, C), jnp.float32),
        ],
    )(vs, coordsT, features)


def _make_sc_scatter(B, C, N):
    BPC = B // NC
    PT = N // NS
    CH = 1024
    NCHUNK = PT // CH
    JROWS = CH // 128
    RT = R_PAD // NS

    mesh = plsc.VectorSubcoreMesh(core_axis_name="c", subcore_axis_name="s")

    @functools.partial(
        pl.kernel,
        out_type=[
            jax.ShapeDtypeStruct((B, R_PAD, C), jnp.float32),
            jax.ShapeDtypeStruct((B, R_PAD, CNT_W), jnp.float32),
        ],
        mesh=mesh,
        compiler_params=pltpu.CompilerParams(use_tc_tiling_on_sc=False),
        scratch_types=[
            tuple(pltpu.VMEM((128,), jnp.int32) for _ in range(JROWS)),
            pltpu.VMEM((CH, C), jnp.float32),
            pltpu.VMEM((128, CNT_W), jnp.float32),
            pltpu.VMEM((64, C), jnp.float32),
            pltpu.VMEM((64, CNT_W), jnp.float32),
            pltpu.VMEM_SHARED((R_PAD, C), jnp.float32),
            pltpu.VMEM_SHARED((R_PAD, CNT_W), jnp.float32),
        ],
    )
    def sc_scatter(featT_hbm, idx_hbm, sums_hbm, cnts_hbm,
                   idx_v, feat_v, ones_v, zf_v, zc_v, acc_s, cnt_s):
        cid = lax.axis_index("c")
        sid = lax.axis_index("s")

        zero16 = jnp.zeros((16,), jnp.float32)
        one16 = jnp.ones((16,), jnp.float32)

        def init_zrow(r, carry):
            for jj in range(C // 16):
                zf_v[r, pl.ds(jj * 16, 16)] = zero16
            zc_v[r, pl.ds(0, CNT_W)] = zero16
            return carry

        lax.fori_loop(0, 64, init_zrow, 0)

        def init_orow(r, carry):
            ones_v[r, pl.ds(0, CNT_W)] = one16
            return carry

        lax.fori_loop(0, 128, init_orow, 0)

        for t in range(BPC):
            b = cid * BPC + t
            row0 = pl.multiple_of(sid * RT, RT)

            def zero_body(z, carry):
                zr = pl.multiple_of(row0 + z * 64, 64)
                pltpu.sync_copy(zf_v, acc_s.at[pl.ds(zr, 64)])
                pltpu.sync_copy(zc_v, cnt_s.at[pl.ds(zr, 64)])
                return carry

            lax.fori_loop(0, RT // 64, zero_body, 0)
            plsc.subcore_barrier()

            def chunk_body(k, carry):
                n0 = pl.multiple_of(sid * PT + k * CH, CH)
                crow = pl.multiple_of(n0 // 128, JROWS)
                for j in range(JROWS):
                    pltpu.sync_copy(idx_hbm.at[b, crow + j], idx_v[j])
                pltpu.sync_copy(featT_hbm.at[b, pl.ds(n0, CH)], feat_v)
                for j in range(JROWS):
                    pltpu.sync_copy(feat_v.at[pl.ds(j * 128, 128)],
                                    acc_s.at[idx_v[j]], add=True)
                    pltpu.sync_copy(ones_v, cnt_s.at[idx_v[j]], add=True)
                return carry

            lax.fori_loop(0, NCHUNK, chunk_body, 0)
            plsc.subcore_barrier()

            pltpu.sync_copy(acc_s.at[pl.ds(row0, RT)],
                            sums_hbm.at[b, pl.ds(row0, RT)])
            pltpu.sync_copy(cnt_s.at[pl.ds(row0, RT)],
                            cnts_hbm.at[b, pl.ds(row0, RT)])

    return sc_scatter


def _postpass_body(sums_ref, cnt_ref, out_ref):
    sm = sums_ref[0]
    ct = cnt_ref[0][:, 0:1]
    avg = jnp.where(ct > 0.0, sm / jnp.maximum(ct, 1.0), 0.0)
    out_ref[0] = avg.T


def _postpass(sums, cnts, B, C):
    return pl.pallas_call(
        _postpass_body,
        grid=(B,),
        in_specs=[
            pl.BlockSpec((1, R, C), lambda b: (b, 0, 0)),
            pl.BlockSpec((1, R, CNT_W), lambda b: (b, 0, 0)),
        ],
        out_specs=pl.BlockSpec((1, C, R), lambda b: (b, 0, 0)),
        out_shape=jax.ShapeDtypeStruct((B, C, R), jnp.float32),
    )(sums, cnts)


def kernel(features, coords, search_area):
    B, C, N = features.shape
    vs = (search_area.astype(jnp.float32) / 20.0)[:, :, None]
    coordsT = jnp.transpose(coords, (0, 2, 1))
    idx3, featT = _prepass(vs, coordsT, features.astype(jnp.float32))
    idxr = idx3.reshape(B, N // 128, 128)
    sums, cnts = _make_sc_scatter(B, C, N)(featT, idxr)
    return _postpass(sums, cnts, B, C)


# speedup vs baseline: 5.6108x; 2.0772x over previous
import functools

import jax
import jax.numpy as jnp
from jax import lax
from jax.experimental import pallas as pl
from jax.experimental.pallas import tpu as pltpu
from jax.experimental.pallas import tpu_sc as plsc

XD, YD, ZD = 20, 20, 20
R = XD * YD * ZD
R_PAD = 8192
NC, NS = 2, 16
CNT_W = 16


def _quantize(c, vs):
    d = jnp.floor(c / vs)
    vi = jnp.clip(d + 10.0, 0.0, 19.0)
    flat = vi[0:1] * float(YD * ZD) + vi[1:2] * float(ZD) + vi[2:3]
    return flat.astype(jnp.int32)


def _prepass_body(vs_ref, ca_ref, cb_ref, fa_ref, fb_ref,
                  idxa_ref, idxb_ref, featp_ref):
    vs = vs_ref[0]
    idxa_ref[0] = _quantize(ca_ref[0], vs)
    idxb_ref[0] = _quantize(cb_ref[0], vs)
    featp_ref[0] = jnp.concatenate([fa_ref[0].T, fb_ref[0].T], axis=1)


def _prepass(vs, coordsT, features):
    B, C, N = features.shape
    NH = N // 2
    NB = 4096
    OFF = NH // NB
    return pl.pallas_call(
        _prepass_body,
        grid=(B, OFF),
        in_specs=[
            pl.BlockSpec((1, 3, 1), lambda b, i: (b, 0, 0)),
            pl.BlockSpec((1, 3, NB), lambda b, i: (b, 0, i)),
            pl.BlockSpec((1, 3, NB), lambda b, i: (b, 0, i + OFF)),
            pl.BlockSpec((1, C, NB), lambda b, i: (b, 0, i)),
            pl.BlockSpec((1, C, NB), lambda b, i: (b, 0, i + OFF)),
        ],
        out_specs=[
            pl.BlockSpec((1, 1, NB), lambda b, i: (b, 0, i)),
            pl.BlockSpec((1, 1, NB), lambda b, i: (b, 0, i)),
            pl.BlockSpec((1, NB, 2 * C), lambda b, i: (b, i, 0)),
        ],
        out_shape=[
            jax.ShapeDtypeStruct((B, 1, NH), jnp.int32),
            jax.ShapeDtypeStruct((B, 1, NH), jnp.int32),
            jax.ShapeDtypeStruct((B, NH, 2 * C), jnp.float32),
        ],
    )(vs, coordsT, coordsT, features, features)


def _make_sc_scatter(B, C, N):
    BPC = B // NC
    PT = N // NS
    CH = 512
    NCHUNK = PT // CH
    JROWS = CH // 128
    RT = R_PAD // NS
    ZR = 128

    mesh = plsc.VectorSubcoreMesh(core_axis_name="c", subcore_axis_name="s")

    @functools.partial(
        pl.kernel,
        out_type=[
            jax.ShapeDtypeStruct((B, R_PAD, C), jnp.float32),
            jax.ShapeDtypeStruct((B, R_PAD, CNT_W), jnp.float32),
        ],
        mesh=mesh,
        compiler_params=pltpu.CompilerParams(use_tc_tiling_on_sc=False,
                                             needs_layout_passes=False),
        scratch_types=[
            tuple(pltpu.VMEM((128,), jnp.int32) for _ in range(2 * JROWS)),
            tuple(pltpu.VMEM((CH // 2,), jnp.int32) for _ in range(2)),
            tuple(pltpu.VMEM((CH // 2,), jnp.int32) for _ in range(2)),
            tuple(pltpu.VMEM((CH, C), jnp.float32) for _ in range(2)),
            pltpu.VMEM((128, CNT_W), jnp.float32),
            pltpu.VMEM((ZR, C), jnp.float32),
            pltpu.VMEM((ZR, CNT_W), jnp.float32),
            pltpu.SemaphoreType.DMA,
            pltpu.SemaphoreType.DMA,
            pltpu.SemaphoreType.DMA,
            pltpu.VMEM_SHARED((R_PAD, C), jnp.float32),
            pltpu.VMEM_SHARED((R_PAD, CNT_W), jnp.float32),
        ],
    )
    def sc_scatter(featT_hbm, idxa_hbm, idxb_hbm, sums_hbm, cnts_hbm,
                   idx_v, ia_v, ib_v, feat_v, ones_v, zf_v, zc_v,
                   sem0, sem1, semz, acc_s, cnt_s):
        cid = lax.axis_index("c")
        sid = lax.axis_index("s")
        sems = (sem0, sem1)
        iota16 = lax.iota(jnp.int32, 16)
        half16 = iota16 >> 1
        even16 = (iota16 & 1) == 0

        zero16 = jnp.zeros((16,), jnp.float32)
        one16 = jnp.ones((16,), jnp.float32)

        def init_zrow(r, carry):
            for jj in range(C // 16):
                zf_v[r, pl.ds(jj * 16, 16)] = zero16
            zc_v[r, pl.ds(0, CNT_W)] = zero16
            return carry

        lax.fori_loop(0, ZR, init_zrow, 0)

        def init_orow(r, carry):
            ones_v[r, pl.ds(0, CNT_W)] = one16
            return carry

        lax.fori_loop(0, 128, init_orow, 0)

        def chunk_copies(b, k, par):
            n0 = pl.multiple_of(sid * PT + k * CH, CH)
            q0 = pl.multiple_of(n0 // 2, CH // 2)
            return [
                pltpu.make_async_copy(
                    featT_hbm.at[b, pl.ds(n0, CH)], feat_v[par], sems[par]),
                pltpu.make_async_copy(
                    idxa_hbm.at[b, pl.ds(q0, CH // 2)], ia_v[par], sems[par]),
                pltpu.make_async_copy(
                    idxb_hbm.at[b, pl.ds(q0, CH // 2)], ib_v[par], sems[par]),
            ]

        def interleave_idx(par):
            for j in range(JROWS):
                dst = idx_v[par * JROWS + j]
                for gg in range(8):
                    src = half16 + (64 * j + 8 * gg)
                    av = plsc.load_gather(ia_v[par], [src])
                    bv = plsc.load_gather(ib_v[par], [src])
                    dst[pl.ds(16 * gg, 16)] = jnp.where(even16, av, bv)

        for t in range(BPC):
            b = cid * BPC + t
            row0 = pl.multiple_of(sid * RT, RT)

            for cp in chunk_copies(b, 0, 0):
                cp.start()
            zcopies = []
            for z in range(RT // ZR):
                zr = pl.multiple_of(row0 + z * ZR, ZR)
                zcopies.append(pltpu.make_async_copy(
                    zf_v, acc_s.at[pl.ds(zr, ZR)], semz))
                zcopies.append(pltpu.make_async_copy(
                    zc_v, cnt_s.at[pl.ds(zr, ZR)], semz))
            for cp in zcopies:
                cp.start()
            for cp in zcopies:
                cp.wait()
            plsc.subcore_barrier()

            def pair_body(g, carry):
                for par in range(2):
                    k = 2 * g + par

                    @pl.when(k + 1 < NCHUNK)
                    def _():
                        for cp in chunk_copies(b, k + 1, 1 - par):
                            cp.start()

                    for cp in chunk_copies(b, k, par):
                        cp.wait()
                    interleave_idx(par)
                    for j in range(JROWS):
                        row = idx_v[par * JROWS + j]
                        pltpu.sync_copy(feat_v[par].at[pl.ds(j * 128, 128)],
                                        acc_s.at[row], add=True)
                        pltpu.sync_copy(ones_v, cnt_s.at[row], add=True)
                return carry

            lax.fori_loop(0, NCHUNK // 2, pair_body, 0)
            plsc.subcore_barrier()

            pltpu.sync_copy(acc_s.at[pl.ds(row0, RT)],
                            sums_hbm.at[b, pl.ds(row0, RT)])
            pltpu.sync_copy(cnt_s.at[pl.ds(row0, RT)],
                            cnts_hbm.at[b, pl.ds(row0, RT)])

    return sc_scatter


def _postpass_body(sums_ref, cnt_ref, out_ref):
    sm = sums_ref[0]
    ct = cnt_ref[0][:, 0:1]
    avg = jnp.where(ct > 0.0, sm / jnp.maximum(ct, 1.0), 0.0)
    out_ref[0] = avg.T


def _postpass(sums, cnts, B, C):
    return pl.pallas_call(
        _postpass_body,
        grid=(B,),
        in_specs=[
            pl.BlockSpec((1, R, C), lambda b: (b, 0, 0)),
            pl.BlockSpec((1, R, CNT_W), lambda b: (b, 0, 0)),
        ],
        out_specs=pl.BlockSpec((1, C, R), lambda b: (b, 0, 0)),
        out_shape=jax.ShapeDtypeStruct((B, C, R), jnp.float32),
    )(sums, cnts)


def kernel(features, coords, search_area):
    B, C, N = features.shape
    vs = (search_area.astype(jnp.float32) / 20.0)[:, :, None]
    coordsT = jnp.transpose(coords, (0, 2, 1))
    idxa, idxb, featp = _prepass(vs, coordsT, features.astype(jnp.float32))
    featT = featp.reshape(B, N, C)
    idxa2 = idxa.reshape(B, N // 2)
    idxb2 = idxb.reshape(B, N // 2)
    sums, cnts = _make_sc_scatter(B, C, N)(featT, idxa2, idxb2)
    return _postpass(sums, cnts, B, C)


# speedup vs baseline: 6.2211x; 1.1088x over previous
import functools

import jax
import jax.numpy as jnp
from jax import lax
from jax.experimental import pallas as pl
from jax.experimental.pallas import tpu as pltpu
from jax.experimental.pallas import tpu_sc as plsc

XD, YD, ZD = 20, 20, 20
R = XD * YD * ZD
R_PAD = 8192
NC, NS = 2, 16
CNT_W = 16


def _quantize(c, vs):
    d = jnp.floor(c / vs)
    vi = jnp.clip(d + 10.0, 0.0, 19.0)
    flat = vi[0:1] * float(YD * ZD) + vi[1:2] * float(ZD) + vi[2:3]
    return flat.astype(jnp.int32)


def _prepass_body(vs_ref, ca_ref, cb_ref, fa_ref, fb_ref,
                  idxa_ref, idxb_ref, featp_ref):
    vs = vs_ref[0]
    idxa_ref[0] = _quantize(ca_ref[0], vs)
    idxb_ref[0] = _quantize(cb_ref[0], vs)
    featp_ref[0] = jnp.concatenate([fa_ref[0].T, fb_ref[0].T], axis=1)


def _prepass(vs, coordsT, features):
    B, C, N = features.shape
    NH = N // 2
    NB = 4096
    OFF = NH // NB
    return pl.pallas_call(
        _prepass_body,
        grid=(B, OFF),
        in_specs=[
            pl.BlockSpec((1, 3, 1), lambda b, i: (b, 0, 0)),
            pl.BlockSpec((1, 3, NB), lambda b, i: (b, 0, i)),
            pl.BlockSpec((1, 3, NB), lambda b, i: (b, 0, i + OFF)),
            pl.BlockSpec((1, C, NB), lambda b, i: (b, 0, i)),
            pl.BlockSpec((1, C, NB), lambda b, i: (b, 0, i + OFF)),
        ],
        out_specs=[
            pl.BlockSpec((1, 1, NB), lambda b, i: (b, 0, i)),
            pl.BlockSpec((1, 1, NB), lambda b, i: (b, 0, i)),
            pl.BlockSpec((1, NB, 2 * C), lambda b, i: (b, i, 0)),
        ],
        out_shape=[
            jax.ShapeDtypeStruct((B, 1, NH), jnp.int32),
            jax.ShapeDtypeStruct((B, 1, NH), jnp.int32),
            jax.ShapeDtypeStruct((B, NH, 2 * C), jnp.float32),
        ],
    )(vs, coordsT, coordsT, features, features)


def _make_sc_scatter(B, C, N):
    BPC = B // NC
    PT = N // NS
    CH = 512
    NCHUNK = PT // CH
    JROWS = CH // 128
    RT = R_PAD // NS
    ZR = 128

    mesh = plsc.VectorSubcoreMesh(core_axis_name="c", subcore_axis_name="s")

    @functools.partial(
        pl.kernel,
        out_type=[
            jax.ShapeDtypeStruct((B, R_PAD, 128), jnp.float32),
        ],
        mesh=mesh,
        compiler_params=pltpu.CompilerParams(use_tc_tiling_on_sc=False,
                                             needs_layout_passes=False),
        scratch_types=[
            tuple(pltpu.VMEM((128,), jnp.int32) for _ in range(2 * JROWS)),
            tuple(pltpu.VMEM((CH // 2,), jnp.int32) for _ in range(2)),
            tuple(pltpu.VMEM((CH // 2,), jnp.int32) for _ in range(2)),
            tuple(pltpu.VMEM((CH, C), jnp.float32) for _ in range(2)),
            pltpu.VMEM((128, CNT_W), jnp.float32),
            pltpu.VMEM((ZR, C), jnp.float32),
            pltpu.VMEM((ZR, CNT_W), jnp.float32),
            pltpu.SemaphoreType.DMA,
            pltpu.SemaphoreType.DMA,
            pltpu.SemaphoreType.DMA,
            pltpu.VMEM_SHARED((R_PAD, C), jnp.float32),
            pltpu.VMEM_SHARED((R_PAD, CNT_W), jnp.float32),
        ],
    )
    def sc_scatter(featT_hbm, idxa_hbm, idxb_hbm, out_hbm,
                   idx_v, ia_v, ib_v, feat_v, ones_v, zf_v, zc_v,
                   sem0, sem1, semz, acc_s, cnt_s):
        cid = lax.axis_index("c")
        sid = lax.axis_index("s")
        sems = (sem0, sem1)
        iota16 = lax.iota(jnp.int32, 16)
        half16 = iota16 >> 1
        even16 = (iota16 & 1) == 0

        zero16 = jnp.zeros((16,), jnp.float32)
        one16 = jnp.ones((16,), jnp.float32)

        def init_zrow(r, carry):
            for jj in range(C // 16):
                zf_v[r, pl.ds(jj * 16, 16)] = zero16
            zc_v[r, pl.ds(0, CNT_W)] = zero16
            return carry

        lax.fori_loop(0, ZR, init_zrow, 0)

        def init_orow(r, carry):
            ones_v[r, pl.ds(0, CNT_W)] = one16
            return carry

        lax.fori_loop(0, 128, init_orow, 0)

        def chunk_copies(b, k, par):
            n0 = pl.multiple_of(sid * PT + k * CH, CH)
            q0 = pl.multiple_of(n0 // 2, CH // 2)
            return [
                pltpu.make_async_copy(
                    featT_hbm.at[b, pl.ds(n0, CH)], feat_v[par], sems[par]),
                pltpu.make_async_copy(
                    idxa_hbm.at[b, pl.ds(q0, CH // 2)], ia_v[par], sems[par]),
                pltpu.make_async_copy(
                    idxb_hbm.at[b, pl.ds(q0, CH // 2)], ib_v[par], sems[par]),
            ]

        def interleave_idx(par):
            for j in range(JROWS):
                dst = idx_v[par * JROWS + j]
                for gg in range(8):
                    src = half16 + (64 * j + 8 * gg)
                    av = plsc.load_gather(ia_v[par], [src])
                    bv = plsc.load_gather(ib_v[par], [src])
                    dst[pl.ds(16 * gg, 16)] = jnp.where(even16, av, bv)

        for t in range(BPC):
            b = cid * BPC + t
            row0 = pl.multiple_of(sid * RT, RT)

            for cp in chunk_copies(b, 0, 0):
                cp.start()
            zcopies = []
            for z in range(RT // ZR):
                zr = pl.multiple_of(row0 + z * ZR, ZR)
                zcopies.append(pltpu.make_async_copy(
                    zf_v, acc_s.at[pl.ds(zr, ZR)], semz))
                zcopies.append(pltpu.make_async_copy(
                    zc_v, cnt_s.at[pl.ds(zr, ZR)], semz))
            for cp in zcopies:
                cp.start()
            for cp in zcopies:
                cp.wait()
            plsc.subcore_barrier()

            def pair_body(g, carry):
                for par in range(2):
                    k = 2 * g + par

                    @pl.when(k + 1 < NCHUNK)
                    def _():
                        for cp in chunk_copies(b, k + 1, 1 - par):
                            cp.start()

                    for cp in chunk_copies(b, k, par):
                        cp.wait()
                    interleave_idx(par)
                    for j in range(JROWS):
                        row = idx_v[par * JROWS + j]
                        pltpu.sync_copy(feat_v[par].at[pl.ds(j * 128, 128)],
                                        acc_s.at[row], add=True)
                        pltpu.sync_copy(ones_v, cnt_s.at[row], add=True)
                return carry

            lax.fori_loop(0, NCHUNK // 2, pair_body, 0)
            plsc.subcore_barrier()

            pltpu.sync_copy(acc_s.at[pl.ds(row0, RT)],
                            out_hbm.at[b, pl.ds(row0, RT), pl.ds(0, C)])
            pltpu.sync_copy(cnt_s.at[pl.ds(row0, RT)],
                            out_hbm.at[b, pl.ds(row0, RT), pl.ds(C, CNT_W)])

    return sc_scatter


def _postpass_body(acc_ref, out_ref):
    blk = acc_ref[0]
    sm = blk[:, 0:64]
    ct = blk[:, 64:65]
    avg = jnp.where(ct > 0.0, sm / jnp.maximum(ct, 1.0), 0.0)
    out_ref[0] = avg.T


def _postpass(acc, B, C):
    return pl.pallas_call(
        _postpass_body,
        grid=(B,),
        in_specs=[
            pl.BlockSpec((1, R, 128), lambda b: (b, 0, 0)),
        ],
        out_specs=pl.BlockSpec((1, C, R), lambda b: (b, 0, 0)),
        out_shape=jax.ShapeDtypeStruct((B, C, R), jnp.float32),
    )(acc)


def kernel(features, coords, search_area):
    B, C, N = features.shape
    vs = (search_area.astype(jnp.float32) / 20.0)[:, :, None]
    coordsT = jnp.transpose(coords, (0, 2, 1))
    idxa, idxb, featp = _prepass(vs, coordsT, features.astype(jnp.float32))
    featT = featp.reshape(B, N, C)
    idxa2 = idxa.reshape(B, N // 2)
    idxb2 = idxb.reshape(B, N // 2)
    (acc,) = _make_sc_scatter(B, C, N)(featT, idxa2, idxb2)
    return _postpass(acc, B, C)


# speedup vs baseline: 8.2037x; 1.3187x over previous
import functools

import jax
import jax.numpy as jnp
from jax import lax
from jax.experimental import pallas as pl
from jax.experimental.pallas import tpu as pltpu
from jax.experimental.pallas import tpu_sc as plsc

XD, YD, ZD = 20, 20, 20
R = XD * YD * ZD
R_PAD = 8192
NC, NS = 2, 16
CNT_W = 16


def _quantize(c, vs):
    d = jnp.floor(c / vs)
    vi = jnp.clip(d + 10.0, 0.0, 19.0)
    flat = vi[0:1] * float(YD * ZD) + vi[1:2] * float(ZD) + vi[2:3]
    return flat.astype(jnp.int32)


def _prepass_body(vs_ref, ca_ref, cb_ref, fa_ref, fb_ref,
                  idxa_ref, idxb_ref, featp_ref):
    vs = vs_ref[0]
    idxa_ref[0] = _quantize(ca_ref[0], vs)
    idxb_ref[0] = _quantize(cb_ref[0], vs)
    featp_ref[0] = jnp.concatenate([fa_ref[0].T, fb_ref[0].T], axis=1)


def _prepass(vs, coordsT, features, b0, bg):
    B, C, N = features.shape
    NH = N // 2
    NB = 4096
    OFF = NH // NB
    return pl.pallas_call(
        _prepass_body,
        grid=(bg, OFF),
        in_specs=[
            pl.BlockSpec((1, 3, 1), lambda b, i: (b + b0, 0, 0)),
            pl.BlockSpec((1, 3, NB), lambda b, i: (b + b0, 0, i)),
            pl.BlockSpec((1, 3, NB), lambda b, i: (b + b0, 0, i + OFF)),
            pl.BlockSpec((1, C, NB), lambda b, i: (b + b0, 0, i)),
            pl.BlockSpec((1, C, NB), lambda b, i: (b + b0, 0, i + OFF)),
        ],
        out_specs=[
            pl.BlockSpec((1, 1, NB), lambda b, i: (b, 0, i)),
            pl.BlockSpec((1, 1, NB), lambda b, i: (b, 0, i)),
            pl.BlockSpec((1, NB, 2 * C), lambda b, i: (b, i, 0)),
        ],
        out_shape=[
            jax.ShapeDtypeStruct((bg, 1, NH), jnp.int32),
            jax.ShapeDtypeStruct((bg, 1, NH), jnp.int32),
            jax.ShapeDtypeStruct((bg, NH, 2 * C), jnp.float32),
        ],
    )(vs, coordsT, coordsT, features, features)


def _make_sc_scatter(B, C, N):
    BPC = B // NC
    PT = N // NS
    CH = 512
    NCHUNK = PT // CH
    JROWS = CH // 128
    RT = R_PAD // NS
    ZR = 128

    mesh = plsc.VectorSubcoreMesh(core_axis_name="c", subcore_axis_name="s")

    @functools.partial(
        pl.kernel,
        out_type=[
            jax.ShapeDtypeStruct((B, R_PAD, 128), jnp.float32),
        ],
        mesh=mesh,
        compiler_params=pltpu.CompilerParams(use_tc_tiling_on_sc=False,
                                             needs_layout_passes=False),
        scratch_types=[
            tuple(pltpu.VMEM((128,), jnp.int32) for _ in range(2 * JROWS)),
            tuple(pltpu.VMEM((CH // 2,), jnp.int32) for _ in range(2)),
            tuple(pltpu.VMEM((CH // 2,), jnp.int32) for _ in range(2)),
            tuple(pltpu.VMEM((CH, C), jnp.float32) for _ in range(2)),
            pltpu.VMEM((128, CNT_W), jnp.float32),
            pltpu.VMEM((ZR, C), jnp.float32),
            pltpu.VMEM((ZR, CNT_W), jnp.float32),
            pltpu.SemaphoreType.DMA,
            pltpu.SemaphoreType.DMA,
            pltpu.SemaphoreType.DMA,
            pltpu.VMEM_SHARED((R_PAD, C), jnp.float32),
            pltpu.VMEM_SHARED((R_PAD, CNT_W), jnp.float32),
        ],
    )
    def sc_scatter(featT_hbm, idxa_hbm, idxb_hbm, out_hbm,
                   idx_v, ia_v, ib_v, feat_v, ones_v, zf_v, zc_v,
                   sem0, sem1, semz, acc_s, cnt_s):
        cid = lax.axis_index("c")
        sid = lax.axis_index("s")
        sems = (sem0, sem1)
        iota16 = lax.iota(jnp.int32, 16)
        half16 = iota16 >> 1
        even16 = (iota16 & 1) == 0

        zero16 = jnp.zeros((16,), jnp.float32)
        one16 = jnp.ones((16,), jnp.float32)

        def init_zrow(r, carry):
            for jj in range(C // 16):
                zf_v[r, pl.ds(jj * 16, 16)] = zero16
            zc_v[r, pl.ds(0, CNT_W)] = zero16
            return carry

        lax.fori_loop(0, ZR, init_zrow, 0)

        def init_orow(r, carry):
            ones_v[r, pl.ds(0, CNT_W)] = one16
            return carry

        lax.fori_loop(0, 128, init_orow, 0)

        def chunk_copies(b, k, par):
            n0 = pl.multiple_of(sid * PT + k * CH, CH)
            q0 = pl.multiple_of(n0 // 2, CH // 2)
            return [
                pltpu.make_async_copy(
                    featT_hbm.at[b, pl.ds(n0, CH)], feat_v[par], sems[par]),
                pltpu.make_async_copy(
                    idxa_hbm.at[b, pl.ds(q0, CH // 2)], ia_v[par], sems[par]),
                pltpu.make_async_copy(
                    idxb_hbm.at[b, pl.ds(q0, CH // 2)], ib_v[par], sems[par]),
            ]

        def interleave_idx(par):
            for j in range(JROWS):
                dst = idx_v[par * JROWS + j]
                for gg in range(8):
                    src = half16 + (64 * j + 8 * gg)
                    av = plsc.load_gather(ia_v[par], [src])
                    bv = plsc.load_gather(ib_v[par], [src])
                    dst[pl.ds(16 * gg, 16)] = jnp.where(even16, av, bv)

        for t in range(BPC):
            b = cid * BPC + t
            row0 = pl.multiple_of(sid * RT, RT)

            for cp in chunk_copies(b, 0, 0):
                cp.start()
            zcopies = []
            for z in range(RT // ZR):
                zr = pl.multiple_of(row0 + z * ZR, ZR)
                zcopies.append(pltpu.make_async_copy(
                    zf_v, acc_s.at[pl.ds(zr, ZR)], semz))
                zcopies.append(pltpu.make_async_copy(
                    zc_v, cnt_s.at[pl.ds(zr, ZR)], semz))
            for cp in zcopies:
                cp.start()
            for cp in zcopies:
                cp.wait()
            plsc.subcore_barrier()

            def pair_body(g, carry):
                for par in range(2):
                    k = 2 * g + par

                    @pl.when(k + 1 < NCHUNK)
                    def _():
                        for cp in chunk_copies(b, k + 1, 1 - par):
                            cp.start()

                    for cp in chunk_copies(b, k, par):
                        cp.wait()
                    interleave_idx(par)
                    for j in range(JROWS):
                        row = idx_v[par * JROWS + j]
                        pltpu.sync_copy(feat_v[par].at[pl.ds(j * 128, 128)],
                                        acc_s.at[row], add=True)
                        pltpu.sync_copy(ones_v, cnt_s.at[row], add=True)
                return carry

            lax.fori_loop(0, NCHUNK // 2, pair_body, 0)
            plsc.subcore_barrier()

            pltpu.sync_copy(acc_s.at[pl.ds(row0, RT)],
                            out_hbm.at[b, pl.ds(row0, RT), pl.ds(0, C)])
            pltpu.sync_copy(cnt_s.at[pl.ds(row0, RT)],
                            out_hbm.at[b, pl.ds(row0, RT), pl.ds(C, CNT_W)])

    return sc_scatter


def _postpass_body(acc_ref, out_ref):
    blk = acc_ref[0]
    sm = blk[:, 0:64]
    ct = blk[:, 64:65]
    avg = jnp.where(ct > 0.0, sm / jnp.maximum(ct, 1.0), 0.0)
    out_ref[0] = avg.T


def _postpass(acc, B, C):
    return pl.pallas_call(
        _postpass_body,
        grid=(B,),
        in_specs=[
            pl.BlockSpec((1, R, 128), lambda b: (b, 0, 0)),
        ],
        out_specs=pl.BlockSpec((1, C, R), lambda b: (b, 0, 0)),
        out_shape=jax.ShapeDtypeStruct((B, C, R), jnp.float32),
    )(acc)


def kernel(features, coords, search_area):
    B, C, N = features.shape
    vs = (search_area.astype(jnp.float32) / 20.0)[:, :, None]
    coordsT = jnp.transpose(coords, (0, 2, 1))
    BG = 2
    features = features.astype(jnp.float32)
    sc_call = _make_sc_scatter(BG, C, N)
    outs = []
    for b0 in range(0, B, BG):
        idxa, idxb, featp = _prepass(vs, coordsT, features, b0, BG)
        featT = featp.reshape(BG, N, C)
        idxa2 = idxa.reshape(BG, N // 2)
        idxb2 = idxb.reshape(BG, N // 2)
        (acc,) = sc_call(featT, idxa2, idxb2)
        outs.append(_postpass(acc, BG, C))
    return jnp.concatenate(outs, axis=0)


# speedup vs baseline: 8.2414x; 1.0046x over previous
import functools

import jax
import jax.numpy as jnp
from jax import lax
from jax.experimental import pallas as pl
from jax.experimental.pallas import tpu as pltpu
from jax.experimental.pallas import tpu_sc as plsc

XD, YD, ZD = 20, 20, 20
R = XD * YD * ZD
R_PAD = 8192
NC, NS = 2, 16
CNT_W = 16


def _quantize(c, vs):
    d = jnp.floor(c / vs)
    vi = jnp.clip(d + 10.0, 0.0, 19.0)
    flat = vi[0:1] * float(YD * ZD) + vi[1:2] * float(ZD) + vi[2:3]
    return flat.astype(jnp.int32)


def _prepass_body(vs_ref, ca_ref, cb_ref, fa_ref, fb_ref,
                  idxa_ref, idxb_ref, featp_ref):
    vs = vs_ref[0]
    idxa_ref[0] = _quantize(ca_ref[0], vs)
    idxb_ref[0] = _quantize(cb_ref[0], vs)
    featp_ref[0] = jnp.concatenate([fa_ref[0].T, fb_ref[0].T], axis=1)


def _prepass(vs, coordsT, features, b0, bg):
    B, C, N = features.shape
    NH = N // 2
    NB = 4096
    OFF = NH // NB
    return pl.pallas_call(
        _prepass_body,
        grid=(bg, OFF),
        in_specs=[
            pl.BlockSpec((1, 3, 1), lambda b, i: (b, 0, 0)),
            pl.BlockSpec((1, 3, NB), lambda b, i: (b, 0, i)),
            pl.BlockSpec((1, 3, NB), lambda b, i: (b, 0, i + OFF)),
            pl.BlockSpec((1, C, NB), lambda b, i: (b + b0, 0, i)),
            pl.BlockSpec((1, C, NB), lambda b, i: (b + b0, 0, i + OFF)),
        ],
        out_specs=[
            pl.BlockSpec((1, 1, NB), lambda b, i: (b, 0, i)),
            pl.BlockSpec((1, 1, NB), lambda b, i: (b, 0, i)),
            pl.BlockSpec((1, NB, 2 * C), lambda b, i: (b, i, 0)),
        ],
        out_shape=[
            jax.ShapeDtypeStruct((bg, 1, NH), jnp.int32),
            jax.ShapeDtypeStruct((bg, 1, NH), jnp.int32),
            jax.ShapeDtypeStruct((bg, NH, 2 * C), jnp.float32),
        ],
    )(vs, coordsT, coordsT, features, features)


def _make_sc_scatter(B, C, N):
    BPC = B // NC
    PT = N // NS
    CH = 512
    NCHUNK = PT // CH
    JROWS = CH // 128
    RT = R_PAD // NS
    ZR = 128

    mesh = plsc.VectorSubcoreMesh(core_axis_name="c", subcore_axis_name="s")

    @functools.partial(
        pl.kernel,
        out_type=[
            jax.ShapeDtypeStruct((B, R_PAD, 128), jnp.float32),
        ],
        mesh=mesh,
        compiler_params=pltpu.CompilerParams(use_tc_tiling_on_sc=False,
                                             needs_layout_passes=False),
        scratch_types=[
            tuple(pltpu.VMEM((128,), jnp.int32) for _ in range(2 * JROWS)),
            tuple(pltpu.VMEM((CH // 2,), jnp.int32) for _ in range(2)),
            tuple(pltpu.VMEM((CH // 2,), jnp.int32) for _ in range(2)),
            tuple(pltpu.VMEM((CH, C), jnp.float32) for _ in range(2)),
            pltpu.VMEM((128, CNT_W), jnp.float32),
            pltpu.VMEM((ZR, C), jnp.float32),
            pltpu.VMEM((ZR, CNT_W), jnp.float32),
            pltpu.SemaphoreType.DMA,
            pltpu.SemaphoreType.DMA,
            pltpu.SemaphoreType.DMA,
            pltpu.VMEM_SHARED((R_PAD, C), jnp.float32),
            pltpu.VMEM_SHARED((R_PAD, CNT_W), jnp.float32),
        ],
    )
    def sc_scatter(featT_hbm, idxa_hbm, idxb_hbm, out_hbm,
                   idx_v, ia_v, ib_v, feat_v, ones_v, zf_v, zc_v,
                   sem0, sem1, semz, acc_s, cnt_s):
        cid = lax.axis_index("c")
        sid = lax.axis_index("s")
        sems = (sem0, sem1)
        iota16 = lax.iota(jnp.int32, 16)
        half16 = iota16 >> 1
        even16 = (iota16 & 1) == 0

        zero16 = jnp.zeros((16,), jnp.float32)
        one16 = jnp.ones((16,), jnp.float32)

        def init_zrow(r, carry):
            for jj in range(C // 16):
                zf_v[r, pl.ds(jj * 16, 16)] = zero16
            zc_v[r, pl.ds(0, CNT_W)] = zero16
            return carry

        lax.fori_loop(0, ZR, init_zrow, 0)

        def init_orow(r, carry):
            ones_v[r, pl.ds(0, CNT_W)] = one16
            return carry

        lax.fori_loop(0, 128, init_orow, 0)

        def chunk_copies(b, k, par):
            n0 = pl.multiple_of(sid * PT + k * CH, CH)
            q0 = pl.multiple_of(n0 // 2, CH // 2)
            return [
                pltpu.make_async_copy(
                    featT_hbm.at[b, pl.ds(n0, CH)], feat_v[par], sems[par]),
                pltpu.make_async_copy(
                    idxa_hbm.at[b, pl.ds(q0, CH // 2)], ia_v[par], sems[par]),
                pltpu.make_async_copy(
                    idxb_hbm.at[b, pl.ds(q0, CH // 2)], ib_v[par], sems[par]),
            ]

        def interleave_idx(par):
            for j in range(JROWS):
                dst = idx_v[par * JROWS + j]
                for gg in range(8):
                    src = half16 + (64 * j + 8 * gg)
                    av = plsc.load_gather(ia_v[par], [src])
                    bv = plsc.load_gather(ib_v[par], [src])
                    dst[pl.ds(16 * gg, 16)] = jnp.where(even16, av, bv)

        for t in range(BPC):
            b = cid * BPC + t
            row0 = pl.multiple_of(sid * RT, RT)

            for cp in chunk_copies(b, 0, 0):
                cp.start()
            zcopies = []
            for z in range(RT // ZR):
                zr = pl.multiple_of(row0 + z * ZR, ZR)
                zcopies.append(pltpu.make_async_copy(
                    zf_v, acc_s.at[pl.ds(zr, ZR)], semz))
                zcopies.append(pltpu.make_async_copy(
                    zc_v, cnt_s.at[pl.ds(zr, ZR)], semz))
            for cp in zcopies:
                cp.start()
            for cp in zcopies:
                cp.wait()
            plsc.subcore_barrier()

            def pair_body(g, carry):
                for par in range(2):
                    k = 2 * g + par

                    @pl.when(k + 1 < NCHUNK)
                    def _():
                        for cp in chunk_copies(b, k + 1, 1 - par):
                            cp.start()

                    for cp in chunk_copies(b, k, par):
                        cp.wait()
                    interleave_idx(par)
                    for j in range(JROWS):
                        row = idx_v[par * JROWS + j]
                        pltpu.sync_copy(feat_v[par].at[pl.ds(j * 128, 128)],
                                        acc_s.at[row], add=True)
                        pltpu.sync_copy(ones_v, cnt_s.at[row], add=True)
                return carry

            lax.fori_loop(0, NCHUNK // 2, pair_body, 0)
            plsc.subcore_barrier()

            pltpu.sync_copy(acc_s.at[pl.ds(row0, RT)],
                            out_hbm.at[b, pl.ds(row0, RT), pl.ds(0, C)])
            pltpu.sync_copy(cnt_s.at[pl.ds(row0, RT)],
                            out_hbm.at[b, pl.ds(row0, RT), pl.ds(C, CNT_W)])

    return sc_scatter


def _postpass_body(acc_ref, out_ref):
    blk = acc_ref[0]
    sm = blk[:, 0:64]
    ct = blk[:, 64:65]
    avg = jnp.where(ct > 0.0, sm / jnp.maximum(ct, 1.0), 0.0)
    out_ref[0] = avg.T


def _postpass(acc, B, C):
    return pl.pallas_call(
        _postpass_body,
        grid=(B,),
        in_specs=[
            pl.BlockSpec((1, R, 128), lambda b: (b, 0, 0)),
        ],
        out_specs=pl.BlockSpec((1, C, R), lambda b: (b, 0, 0)),
        out_shape=jax.ShapeDtypeStruct((B, C, R), jnp.float32),
    )(acc)


def kernel(features, coords, search_area):
    B, C, N = features.shape
    vs = (search_area.astype(jnp.float32) / 20.0)[:, :, None]
    coordsT = jnp.transpose(coords, (0, 2, 1))
    BG = 2
    features = features.astype(jnp.float32)
    sc_call = _make_sc_scatter(BG, C, N)
    outs = []
    for b0 in range(0, B, BG):
        coordsT_g = jnp.transpose(coords[b0:b0 + BG], (0, 2, 1))
        idxa, idxb, featp = _prepass(vs[b0:b0 + BG], coordsT_g,
                                     features, b0, BG)
        featT = featp.reshape(BG, N, C)
        idxa2 = idxa.reshape(BG, N // 2)
        idxb2 = idxb.reshape(BG, N // 2)
        (acc,) = sc_call(featT, idxa2, idxb2)
        outs.append(_postpass(acc, BG, C))
    return jnp.concatenate(outs, axis=0)


# speedup vs baseline: 8.2790x; 1.0046x over previous
import functools

import jax
import jax.numpy as jnp
from jax import lax
from jax.experimental import pallas as pl
from jax.experimental.pallas import tpu as pltpu
from jax.experimental.pallas import tpu_sc as plsc

XD, YD, ZD = 20, 20, 20
R = XD * YD * ZD
R_PAD = 8192
NC, NS = 2, 16
CNT_W = 16


def _quantize(c, vs):
    d = jnp.floor(c / vs)
    vi = jnp.clip(d + 10.0, 0.0, 19.0)
    flat = vi[0:1] * float(YD * ZD) + vi[1:2] * float(ZD) + vi[2:3]
    return flat.astype(jnp.int32)


def _prepass_body(vs_ref, ca_ref, cb_ref, fa_ref, fb_ref,
                  idxa_ref, idxb_ref, featp_ref):
    vs = vs_ref[0]
    idxa_ref[0] = _quantize(ca_ref[0], vs)
    idxb_ref[0] = _quantize(cb_ref[0], vs)
    featp_ref[0] = jnp.concatenate([fa_ref[0].T, fb_ref[0].T], axis=1)


def _prepass(vs, coordsT, features, b0, bg):
    B, C, N = features.shape
    NH = N // 2
    NB = 8192
    OFF = NH // NB
    return pl.pallas_call(
        _prepass_body,
        grid=(bg, OFF),
        in_specs=[
            pl.BlockSpec((1, 3, 1), lambda b, i: (b, 0, 0)),
            pl.BlockSpec((1, 3, NB), lambda b, i: (b, 0, i)),
            pl.BlockSpec((1, 3, NB), lambda b, i: (b, 0, i + OFF)),
            pl.BlockSpec((1, C, NB), lambda b, i: (b + b0, 0, i)),
            pl.BlockSpec((1, C, NB), lambda b, i: (b + b0, 0, i + OFF)),
        ],
        out_specs=[
            pl.BlockSpec((1, 1, NB), lambda b, i: (b, 0, i)),
            pl.BlockSpec((1, 1, NB), lambda b, i: (b, 0, i)),
            pl.BlockSpec((1, NB, 2 * C), lambda b, i: (b, i, 0)),
        ],
        out_shape=[
            jax.ShapeDtypeStruct((bg, 1, NH), jnp.int32),
            jax.ShapeDtypeStruct((bg, 1, NH), jnp.int32),
            jax.ShapeDtypeStruct((bg, NH, 2 * C), jnp.float32),
        ],
    )(vs, coordsT, coordsT, features, features)


def _make_sc_scatter(B, C, N):
    BPC = B // NC
    PT = N // NS
    CH = 512
    NCHUNK = PT // CH
    JROWS = CH // 128
    RT = R_PAD // NS
    ZR = 128

    mesh = plsc.VectorSubcoreMesh(core_axis_name="c", subcore_axis_name="s")

    @functools.partial(
        pl.kernel,
        out_type=[
            jax.ShapeDtypeStruct((B, R_PAD, 128), jnp.float32),
        ],
        mesh=mesh,
        compiler_params=pltpu.CompilerParams(use_tc_tiling_on_sc=False,
                                             needs_layout_passes=False),
        scratch_types=[
            tuple(pltpu.VMEM((128,), jnp.int32) for _ in range(2 * JROWS)),
            tuple(pltpu.VMEM((CH // 2,), jnp.int32) for _ in range(2)),
            tuple(pltpu.VMEM((CH // 2,), jnp.int32) for _ in range(2)),
            tuple(pltpu.VMEM((CH, C), jnp.float32) for _ in range(2)),
            pltpu.VMEM((128, CNT_W), jnp.float32),
            pltpu.VMEM((ZR, C), jnp.float32),
            pltpu.VMEM((ZR, CNT_W), jnp.float32),
            pltpu.SemaphoreType.DMA,
            pltpu.SemaphoreType.DMA,
            pltpu.SemaphoreType.DMA,
            pltpu.VMEM_SHARED((R_PAD, C), jnp.float32),
            pltpu.VMEM_SHARED((R_PAD, CNT_W), jnp.float32),
        ],
    )
    def sc_scatter(featT_hbm, idxa_hbm, idxb_hbm, out_hbm,
                   idx_v, ia_v, ib_v, feat_v, ones_v, zf_v, zc_v,
                   sem0, sem1, semz, acc_s, cnt_s):
        cid = lax.axis_index("c")
        sid = lax.axis_index("s")
        sems = (sem0, sem1)
        iota16 = lax.iota(jnp.int32, 16)
        half16 = iota16 >> 1
        even16 = (iota16 & 1) == 0

        zero16 = jnp.zeros((16,), jnp.float32)
        one16 = jnp.ones((16,), jnp.float32)

        def init_zrow(r, carry):
            for jj in range(C // 16):
                zf_v[r, pl.ds(jj * 16, 16)] = zero16
            zc_v[r, pl.ds(0, CNT_W)] = zero16
            return carry

        lax.fori_loop(0, ZR, init_zrow, 0)

        def init_orow(r, carry):
            ones_v[r, pl.ds(0, CNT_W)] = one16
            return carry

        lax.fori_loop(0, 128, init_orow, 0)

        def chunk_copies(b, k, par):
            n0 = pl.multiple_of(sid * PT + k * CH, CH)
            q0 = pl.multiple_of(n0 // 2, CH // 2)
            return [
                pltpu.make_async_copy(
                    featT_hbm.at[b, pl.ds(n0, CH)], feat_v[par], sems[par]),
                pltpu.make_async_copy(
                    idxa_hbm.at[b, pl.ds(q0, CH // 2)], ia_v[par], sems[par]),
                pltpu.make_async_copy(
                    idxb_hbm.at[b, pl.ds(q0, CH // 2)], ib_v[par], sems[par]),
            ]

        def interleave_idx(par):
            for j in range(JROWS):
                dst = idx_v[par * JROWS + j]
                for gg in range(8):
                    src = half16 + (64 * j + 8 * gg)
                    av = plsc.load_gather(ia_v[par], [src])
                    bv = plsc.load_gather(ib_v[par], [src])
                    dst[pl.ds(16 * gg, 16)] = jnp.where(even16, av, bv)

        for t in range(BPC):
            b = cid * BPC + t
            row0 = pl.multiple_of(sid * RT, RT)

            for cp in chunk_copies(b, 0, 0):
                cp.start()
            zcopies = []
            for z in range(RT // ZR):
                zr = pl.multiple_of(row0 + z * ZR, ZR)
                zcopies.append(pltpu.make_async_copy(
                    zf_v, acc_s.at[pl.ds(zr, ZR)], semz))
                zcopies.append(pltpu.make_async_copy(
                    zc_v, cnt_s.at[pl.ds(zr, ZR)], semz))
            for cp in zcopies:
                cp.start()
            for cp in zcopies:
                cp.wait()
            plsc.subcore_barrier()

            def pair_body(g, carry):
                for par in range(2):
                    k = 2 * g + par

                    @pl.when(k + 1 < NCHUNK)
                    def _():
                        for cp in chunk_copies(b, k + 1, 1 - par):
                            cp.start()

                    for cp in chunk_copies(b, k, par):
                        cp.wait()
                    interleave_idx(par)
                    for j in range(JROWS):
                        row = idx_v[par * JROWS + j]
                        pltpu.sync_copy(feat_v[par].at[pl.ds(j * 128, 128)],
                                        acc_s.at[row], add=True)
                        pltpu.sync_copy(ones_v, cnt_s.at[row], add=True)
                return carry

            lax.fori_loop(0, NCHUNK // 2, pair_body, 0)
            plsc.subcore_barrier()

            pltpu.sync_copy(acc_s.at[pl.ds(row0, RT)],
                            out_hbm.at[b, pl.ds(row0, RT), pl.ds(0, C)])
            pltpu.sync_copy(cnt_s.at[pl.ds(row0, RT)],
                            out_hbm.at[b, pl.ds(row0, RT), pl.ds(C, CNT_W)])

    return sc_scatter


def _postpass_body(acc_ref, out_ref):
    blk = acc_ref[0]
    sm = blk[:, 0:64]
    ct = blk[:, 64:65]
    avg = jnp.where(ct > 0.0, sm / jnp.maximum(ct, 1.0), 0.0)
    out_ref[0] = avg.T


def _postpass(acc, B, C):
    return pl.pallas_call(
        _postpass_body,
        grid=(B,),
        in_specs=[
            pl.BlockSpec((1, R, 128), lambda b: (b, 0, 0)),
        ],
        out_specs=pl.BlockSpec((1, C, R), lambda b: (b, 0, 0)),
        out_shape=jax.ShapeDtypeStruct((B, C, R), jnp.float32),
    )(acc)


def kernel(features, coords, search_area):
    B, C, N = features.shape
    vs = (search_area.astype(jnp.float32) / 20.0)[:, :, None]
    coordsT = jnp.transpose(coords, (0, 2, 1))
    BG = 2
    features = features.astype(jnp.float32)
    sc_call = _make_sc_scatter(BG, C, N)
    outs = []
    for b0 in range(0, B, BG):
        coordsT_g = jnp.transpose(coords[b0:b0 + BG], (0, 2, 1))
        idxa, idxb, featp = _prepass(vs[b0:b0 + BG], coordsT_g,
                                     features, b0, BG)
        featT = featp.reshape(BG, N, C)
        idxa2 = idxa.reshape(BG, N // 2)
        idxb2 = idxb.reshape(BG, N // 2)
        (acc,) = sc_call(featT, idxa2, idxb2)
        outs.append(_postpass(acc, BG, C))
    return jnp.concatenate(outs, axis=0)


# speedup vs baseline: 8.6061x; 1.0395x over previous
import functools

import jax
import jax.numpy as jnp
from jax import lax
from jax.experimental import pallas as pl
from jax.experimental.pallas import tpu as pltpu
from jax.experimental.pallas import tpu_sc as plsc

XD, YD, ZD = 20, 20, 20
R = XD * YD * ZD
R_PAD = 8192
NC, NS = 2, 16
CNT_W = 16


def _quantize(c, vs):
    d = jnp.floor(c / vs)
    vi = jnp.clip(d + 10.0, 0.0, 19.0)
    flat = vi[0:1] * float(YD * ZD) + vi[1:2] * float(ZD) + vi[2:3]
    return flat.astype(jnp.int32)


def _prepass_body(vs_ref, ca_ref, cb_ref, fa_ref, fb_ref,
                  idxa_ref, idxb_ref, featp_ref):
    vs = vs_ref[0]
    idxa_ref[0] = _quantize(ca_ref[0], vs)
    idxb_ref[0] = _quantize(cb_ref[0], vs)
    featp_ref[0] = jnp.concatenate([fa_ref[0].T, fb_ref[0].T], axis=1)


def _prepass(vs, coordsT, features, b0, bg):
    B, C, N = features.shape
    NH = N // 2
    NB = 8192
    OFF = NH // NB
    return pl.pallas_call(
        _prepass_body,
        grid=(bg, OFF),
        in_specs=[
            pl.BlockSpec((1, 3, 1), lambda b, i: (b, 0, 0)),
            pl.BlockSpec((1, 3, NB), lambda b, i: (b, 0, i)),
            pl.BlockSpec((1, 3, NB), lambda b, i: (b, 0, i + OFF)),
            pl.BlockSpec((1, C, NB), lambda b, i: (b + b0, 0, i)),
            pl.BlockSpec((1, C, NB), lambda b, i: (b + b0, 0, i + OFF)),
        ],
        out_specs=[
            pl.BlockSpec((1, 1, NB), lambda b, i: (b, 0, i)),
            pl.BlockSpec((1, 1, NB), lambda b, i: (b, 0, i)),
            pl.BlockSpec((1, NB, 2 * C), lambda b, i: (b, i, 0)),
        ],
        out_shape=[
            jax.ShapeDtypeStruct((bg, 1, NH), jnp.int32),
            jax.ShapeDtypeStruct((bg, 1, NH), jnp.int32),
            jax.ShapeDtypeStruct((bg, NH, 2 * C), jnp.float32),
        ],
    )(vs, coordsT, coordsT, features, features)


def _make_sc_scatter(B, C, N):
    BPC = B // NC
    PT = N // NS
    CH = 512
    NCHUNK = PT // CH
    JROWS = CH // 128
    RT = R_PAD // NS
    ZR = 128

    mesh = plsc.VectorSubcoreMesh(core_axis_name="c", subcore_axis_name="s")

    @functools.partial(
        pl.kernel,
        out_type=[
            jax.ShapeDtypeStruct((B, R_PAD, 128), jnp.float32),
        ],
        mesh=mesh,
        compiler_params=pltpu.CompilerParams(use_tc_tiling_on_sc=False,
                                             needs_layout_passes=False),
        scratch_types=[
            tuple(pltpu.VMEM((128,), jnp.int32) for _ in range(2 * JROWS)),
            tuple(pltpu.VMEM((CH // 2,), jnp.int32) for _ in range(2)),
            tuple(pltpu.VMEM((CH // 2,), jnp.int32) for _ in range(2)),
            tuple(pltpu.VMEM((CH, C), jnp.float32) for _ in range(2)),
            pltpu.VMEM((128, CNT_W), jnp.float32),
            pltpu.VMEM((ZR, C), jnp.float32),
            pltpu.VMEM((ZR, CNT_W), jnp.float32),
            pltpu.SemaphoreType.DMA,
            pltpu.SemaphoreType.DMA,
            pltpu.SemaphoreType.DMA,
            pltpu.VMEM_SHARED((R_PAD, C), jnp.float32),
            pltpu.VMEM_SHARED((R_PAD, CNT_W), jnp.float32),
        ],
    )
    def sc_scatter(featT_hbm, idxa_hbm, idxb_hbm, out_hbm,
                   idx_v, ia_v, ib_v, feat_v, ones_v, zf_v, zc_v,
                   sem0, sem1, semz, acc_s, cnt_s):
        cid = lax.axis_index("c")
        sid = lax.axis_index("s")
        sems = (sem0, sem1)
        iota16 = lax.iota(jnp.int32, 16)
        half16 = iota16 >> 1
        even16 = (iota16 & 1) == 0

        zero16 = jnp.zeros((16,), jnp.float32)
        one16 = jnp.ones((16,), jnp.float32)

        def init_zrow(r, carry):
            for jj in range(C // 16):
                zf_v[r, pl.ds(jj * 16, 16)] = zero16
            zc_v[r, pl.ds(0, CNT_W)] = zero16
            return carry

        lax.fori_loop(0, ZR, init_zrow, 0)

        def init_orow(r, carry):
            ones_v[r, pl.ds(0, CNT_W)] = one16
            return carry

        lax.fori_loop(0, 128, init_orow, 0)

        def chunk_copies(b, k, par):
            n0 = pl.multiple_of(sid * PT + k * CH, CH)
            q0 = pl.multiple_of(n0 // 2, CH // 2)
            return [
                pltpu.make_async_copy(
                    featT_hbm.at[b, pl.ds(n0, CH)], feat_v[par], sems[par]),
                pltpu.make_async_copy(
                    idxa_hbm.at[b, pl.ds(q0, CH // 2)], ia_v[par], sems[par]),
                pltpu.make_async_copy(
                    idxb_hbm.at[b, pl.ds(q0, CH // 2)], ib_v[par], sems[par]),
            ]

        def interleave_idx(par):
            for j in range(JROWS):
                dst = idx_v[par * JROWS + j]
                for gg in range(8):
                    src = half16 + (64 * j + 8 * gg)
                    av = plsc.load_gather(ia_v[par], [src])
                    bv = plsc.load_gather(ib_v[par], [src])
                    dst[pl.ds(16 * gg, 16)] = jnp.where(even16, av, bv)

        for t in range(BPC):
            b = cid * BPC + t
            row0 = pl.multiple_of(sid * RT, RT)

            for cp in chunk_copies(b, 0, 0):
                cp.start()
            zcopies = []
            for z in range(RT // ZR):
                zr = pl.multiple_of(row0 + z * ZR, ZR)
                zcopies.append(pltpu.make_async_copy(
                    zf_v, acc_s.at[pl.ds(zr, ZR)], semz))
                zcopies.append(pltpu.make_async_copy(
                    zc_v, cnt_s.at[pl.ds(zr, ZR)], semz))
            for cp in zcopies:
                cp.start()
            for cp in zcopies:
                cp.wait()
            plsc.subcore_barrier()

            def pair_body(g, carry):
                for par in range(2):
                    k = 2 * g + par

                    @pl.when(k + 1 < NCHUNK)
                    def _():
                        for cp in chunk_copies(b, k + 1, 1 - par):
                            cp.start()

                    for cp in chunk_copies(b, k, par):
                        cp.wait()
                    interleave_idx(par)
                    for j in range(JROWS):
                        row = idx_v[par * JROWS + j]
                        pltpu.sync_copy(feat_v[par].at[pl.ds(j * 128, 128)],
                                        acc_s.at[row], add=True)
                        pltpu.sync_copy(ones_v, cnt_s.at[row], add=True)
                return carry

            lax.fori_loop(0, NCHUNK // 2, pair_body, 0)
            plsc.subcore_barrier()

            pltpu.sync_copy(acc_s.at[pl.ds(row0, RT)],
                            out_hbm.at[b, pl.ds(row0, RT), pl.ds(0, C)])
            pltpu.sync_copy(cnt_s.at[pl.ds(row0, RT)],
                            out_hbm.at[b, pl.ds(row0, RT), pl.ds(C, CNT_W)])

    return sc_scatter


def _postpass_body(acc_ref, carry_ref, out_ref):
    del carry_ref
    blk = acc_ref[0]
    sm = blk[:, 0:64]
    ct = blk[:, 64:65]
    avg = jnp.where(ct > 0.0, sm / jnp.maximum(ct, 1.0), 0.0)
    out_ref[0] = avg.T


def _postpass(acc, carry, b0, bg, B, C):
    return pl.pallas_call(
        _postpass_body,
        grid=(bg,),
        in_specs=[
            pl.BlockSpec((1, R, 128), lambda b: (b, 0, 0)),
            pl.BlockSpec(memory_space=pl.ANY),
        ],
        out_specs=pl.BlockSpec((1, C, R), lambda b: (b + b0, 0, 0)),
        out_shape=jax.ShapeDtypeStruct((B, C, R), jnp.float32),
        input_output_aliases={1: 0},
    )(acc, carry)


def kernel(features, coords, search_area):
    B, C, N = features.shape
    vs = (search_area.astype(jnp.float32) / 20.0)[:, :, None]
    coordsT = jnp.transpose(coords, (0, 2, 1))
    BG = 2
    features = features.astype(jnp.float32)
    sc_call = _make_sc_scatter(BG, C, N)
    out = jnp.zeros((B, C, R), jnp.float32)
    for b0 in range(0, B, BG):
        coordsT_g = jnp.transpose(coords[b0:b0 + BG], (0, 2, 1))
        idxa, idxb, featp = _prepass(vs[b0:b0 + BG], coordsT_g,
                                     features, b0, BG)
        featT = featp.reshape(BG, N, C)
        idxa2 = idxa.reshape(BG, N // 2)
        idxb2 = idxb.reshape(BG, N // 2)
        (acc,) = sc_call(featT, idxa2, idxb2)
        out = _postpass(acc, out, b0, BG, B, C)
    return out


# speedup vs baseline: 8.6463x; 1.0047x over previous
import functools

import jax
import jax.numpy as jnp
from jax import lax
from jax.experimental import pallas as pl
from jax.experimental.pallas import tpu as pltpu
from jax.experimental.pallas import tpu_sc as plsc

XD, YD, ZD = 20, 20, 20
R = XD * YD * ZD
R_PAD = 8192
NC, NS = 2, 16
CNT_W = 16


def _quantize(c, vs):
    d = jnp.floor(c / vs)
    vi = jnp.clip(d + 10.0, 0.0, 19.0)
    flat = vi[0:1] * float(YD * ZD) + vi[1:2] * float(ZD) + vi[2:3]
    return flat.astype(jnp.int32)


def _prepass_body(vs_ref, ca_ref, cb_ref, fa_ref, fb_ref,
                  idxa_ref, idxb_ref, featp_ref):
    vs = vs_ref[0]
    idxa_ref[0] = _quantize(ca_ref[0], vs)
    idxb_ref[0] = _quantize(cb_ref[0], vs)
    featp_ref[0] = jnp.concatenate([fa_ref[0].T, fb_ref[0].T], axis=1)


def _prepass(vs, coordsT, features, b0, bg):
    B, C, N = features.shape
    NH = N // 2
    NB = 8192
    OFF = NH // NB
    return pl.pallas_call(
        _prepass_body,
        grid=(bg, OFF),
        in_specs=[
            pl.BlockSpec((1, 3, 1), lambda b, i: (b, 0, 0)),
            pl.BlockSpec((1, 3, NB), lambda b, i: (b, 0, i)),
            pl.BlockSpec((1, 3, NB), lambda b, i: (b, 0, i + OFF)),
            pl.BlockSpec((1, C, NB), lambda b, i: (b + b0, 0, i)),
            pl.BlockSpec((1, C, NB), lambda b, i: (b + b0, 0, i + OFF)),
        ],
        out_specs=[
            pl.BlockSpec((1, 1, NB), lambda b, i: (b, 0, i)),
            pl.BlockSpec((1, 1, NB), lambda b, i: (b, 0, i)),
            pl.BlockSpec((1, NB, 2 * C), lambda b, i: (b, i, 0)),
        ],
        out_shape=[
            jax.ShapeDtypeStruct((bg, 1, NH), jnp.int32),
            jax.ShapeDtypeStruct((bg, 1, NH), jnp.int32),
            jax.ShapeDtypeStruct((bg, NH, 2 * C), jnp.float32),
        ],
    )(vs, coordsT, coordsT, features, features)


def _make_sc_scatter(B, C, N):
    BPC = B // NC
    PT = N // NS
    CH = 512
    NCHUNK = PT // CH
    JROWS = CH // 128
    RT = R_PAD // NS
    ZR = 128

    mesh = plsc.VectorSubcoreMesh(core_axis_name="c", subcore_axis_name="s")

    @functools.partial(
        pl.kernel,
        out_type=[
            jax.ShapeDtypeStruct((B, R_PAD, 128), jnp.float32),
        ],
        mesh=mesh,
        compiler_params=pltpu.CompilerParams(use_tc_tiling_on_sc=False,
                                             needs_layout_passes=False),
        scratch_types=[
            tuple(pltpu.VMEM((128,), jnp.int32) for _ in range(2 * JROWS)),
            tuple(pltpu.VMEM((CH // 2,), jnp.int32) for _ in range(2)),
            tuple(pltpu.VMEM((CH // 2,), jnp.int32) for _ in range(2)),
            tuple(pltpu.VMEM((CH, C), jnp.float32) for _ in range(2)),
            pltpu.VMEM((128, CNT_W), jnp.float32),
            pltpu.VMEM((ZR, C), jnp.float32),
            pltpu.VMEM((ZR, CNT_W), jnp.float32),
            pltpu.SemaphoreType.DMA,
            pltpu.SemaphoreType.DMA,
            pltpu.SemaphoreType.DMA,
            pltpu.VMEM_SHARED((R_PAD, C), jnp.float32),
            pltpu.VMEM_SHARED((R_PAD, CNT_W), jnp.float32),
        ],
    )
    def sc_scatter(featT_hbm, idxa_hbm, idxb_hbm, out_hbm,
                   idx_v, ia_v, ib_v, feat_v, ones_v, zf_v, zc_v,
                   sem0, sem1, semz, acc_s, cnt_s):
        cid = lax.axis_index("c")
        sid = lax.axis_index("s")
        sems = (sem0, sem1)
        iota16 = lax.iota(jnp.int32, 16)
        half16 = iota16 >> 1
        even16 = (iota16 & 1) == 0

        zero16 = jnp.zeros((16,), jnp.float32)
        one16 = jnp.ones((16,), jnp.float32)

        def init_zrow(r, carry):
            for jj in range(C // 16):
                zf_v[r, pl.ds(jj * 16, 16)] = zero16
            zc_v[r, pl.ds(0, CNT_W)] = zero16
            return carry

        lax.fori_loop(0, ZR, init_zrow, 0)

        def init_orow(r, carry):
            ones_v[r, pl.ds(0, CNT_W)] = one16
            return carry

        lax.fori_loop(0, 128, init_orow, 0)

        def chunk_copies(b, k, par):
            n0 = pl.multiple_of(sid * PT + k * CH, CH)
            q0 = pl.multiple_of(n0 // 2, CH // 2)
            return [
                pltpu.make_async_copy(
                    featT_hbm.at[b, pl.ds(n0, CH)], feat_v[par], sems[par]),
                pltpu.make_async_copy(
                    idxa_hbm.at[b, pl.ds(q0, CH // 2)], ia_v[par], sems[par]),
                pltpu.make_async_copy(
                    idxb_hbm.at[b, pl.ds(q0, CH // 2)], ib_v[par], sems[par]),
            ]

        def interleave_idx(par):
            for j in range(JROWS):
                dst = idx_v[par * JROWS + j]
                for gg in range(8):
                    src = half16 + (64 * j + 8 * gg)
                    av = plsc.load_gather(ia_v[par], [src])
                    bv = plsc.load_gather(ib_v[par], [src])
                    dst[pl.ds(16 * gg, 16)] = jnp.where(even16, av, bv)

        for t in range(BPC):
            b = cid * BPC + t
            row0 = pl.multiple_of(sid * RT, RT)

            for cp in chunk_copies(b, 0, 0):
                cp.start()
            zcopies = []
            for z in range(RT // ZR):
                zr = pl.multiple_of(row0 + z * ZR, ZR)
                zcopies.append(pltpu.make_async_copy(
                    zf_v, acc_s.at[pl.ds(zr, ZR)], semz))
                zcopies.append(pltpu.make_async_copy(
                    zc_v, cnt_s.at[pl.ds(zr, ZR)], semz))
            for cp in zcopies:
                cp.start()
            for cp in zcopies:
                cp.wait()
            plsc.subcore_barrier()

            def pair_body(g, carry):
                for par in range(2):
                    k = 2 * g + par

                    @pl.when(k + 1 < NCHUNK)
                    def _():
                        for cp in chunk_copies(b, k + 1, 1 - par):
                            cp.start()

                    for cp in chunk_copies(b, k, par):
                        cp.wait()
                    interleave_idx(par)
                    for j in range(JROWS):
                        row = idx_v[par * JROWS + j]
                        pltpu.sync_copy(feat_v[par].at[pl.ds(j * 128, 128)],
                                        acc_s.at[row], add=True)
                        pltpu.sync_copy(ones_v, cnt_s.at[row], add=True)
                return carry

            lax.fori_loop(0, NCHUNK // 2, pair_body, 0)
            plsc.subcore_barrier()

            pltpu.sync_copy(acc_s.at[pl.ds(row0, RT)],
                            out_hbm.at[b, pl.ds(row0, RT), pl.ds(0, C)])
            pltpu.sync_copy(cnt_s.at[pl.ds(row0, RT)],
                            out_hbm.at[b, pl.ds(row0, RT), pl.ds(C, CNT_W)])

    return sc_scatter


def _postpass_body(*refs):
    acc_ref, out_ref = refs[0], refs[-1]
    blk = acc_ref[0]
    sm = blk[:, 0:64]
    ct = blk[:, 64:65]
    avg = jnp.where(ct > 0.0, sm / jnp.maximum(ct, 1.0), 0.0)
    out_ref[0] = avg.T


def _postpass(acc, carry, b0, bg, B, C):
    in_specs = [pl.BlockSpec((1, R, 128), lambda b: (b, 0, 0))]
    args = [acc]
    aliases = {}
    if carry is not None:
        in_specs.append(pl.BlockSpec(memory_space=pl.ANY))
        args.append(carry)
        aliases = {1: 0}
    return pl.pallas_call(
        _postpass_body,
        grid=(bg,),
        in_specs=in_specs,
        out_specs=pl.BlockSpec((1, C, R), lambda b: (b + b0, 0, 0)),
        out_shape=jax.ShapeDtypeStruct((B, C, R), jnp.float32),
        input_output_aliases=aliases,
    )(*args)


def kernel(features, coords, search_area):
    B, C, N = features.shape
    vs = (search_area.astype(jnp.float32) / 20.0)[:, :, None]
    coordsT = jnp.transpose(coords, (0, 2, 1))
    BG = 2
    features = features.astype(jnp.float32)
    sc_call = _make_sc_scatter(BG, C, N)
    out = None
    for b0 in range(0, B, BG):
        coordsT_g = jnp.transpose(coords[b0:b0 + BG], (0, 2, 1))
        idxa, idxb, featp = _prepass(vs[b0:b0 + BG], coordsT_g,
                                     features, b0, BG)
        featT = featp.reshape(BG, N, C)
        idxa2 = idxa.reshape(BG, N // 2)
        idxb2 = idxb.reshape(BG, N // 2)
        (acc,) = sc_call(featT, idxa2, idxb2)
        out = _postpass(acc, out, b0, BG, B, C)
    return out
